# baseline (speedup 1.0000x reference)
import functools

import jax
import jax.numpy as jnp
from jax import lax
from jax.experimental import pallas as pl
from jax.experimental.pallas import tpu as pltpu
from jax.experimental.pallas import tpu_sc as plsc

NC = 2
NS = 16
L = 16
NW = NC * NS

POS_IOU = 0.7
NEG_IOU = 0.3


def _vlog(x):
    bits = plsc.bitcast(x, jnp.int32)
    e = ((bits >> 23) & 0xFF) - 127
    m = plsc.bitcast((bits & 0x007FFFFF) | 0x3F800000, jnp.float32)
    half = m * 0.5
    big = half >= 0.70710678118654752440
    xr = jnp.where(big, half, m) - 1.0
    e = (e + jnp.where(big, 1, 0)).astype(jnp.float32)
    z = xr * xr
    p = jnp.full_like(xr, 7.0376836292e-2)
    for c in (-1.1514610310e-1, 1.1676998740e-1, -1.2420140846e-1,
              1.4249322787e-1, -1.6668057665e-1, 2.0000714765e-1,
              -2.4999993993e-1, 3.3333331174e-1):
        p = p * xr + c
    y = xr * z * p
    y = y + e * -2.12194440e-4
    y = y - 0.5 * z
    return (xr + y) + e * 0.693359375


def _pass1_body(G, PT, NCH,
                gt_h, w_h, h_h, x0_h, y0_h, x1_h, y1_h,
                lab_h, tgt_h, win_h, code_h, cnt_h,
                gt_v, g_x0, g_y0, g_x1, g_y1, g_ab, g_w, g_hh, g_cx, g_cy,
                a_x0, a_y0, a_x1, a_y1,
                lab_v, tgt_v, win_v, code_v, w_v, h_v, cnt_v):
    wid = lax.axis_index("s") * NC + lax.axis_index("c")
    base = wid * PT

    pltpu.sync_copy(gt_h, gt_v)
    pltpu.sync_copy(w_h, w_v)
    pltpu.sync_copy(h_h, h_v)
    pltpu.sync_copy(x0_h.at[pl.ds(base, PT)], a_x0)
    pltpu.sync_copy(y0_h.at[pl.ds(base, PT)], a_y0)
    pltpu.sync_copy(x1_h.at[pl.ds(base, PT)], a_x1)
    pltpu.sync_copy(y1_h.at[pl.ds(base, PT)], a_y1)

    lanes = lax.iota(jnp.int32, 16)

    for c in range(G // L):
        rows4 = (lanes + c * L) * 4
        bx0 = plsc.load_gather(gt_v, [rows4])
        by0 = plsc.load_gather(gt_v, [rows4 + 1])
        bx1 = plsc.load_gather(gt_v, [rows4 + 2])
        by1 = plsc.load_gather(gt_v, [rows4 + 3])
        gw = bx1 - bx0
        gh = by1 - by0
        sl = pl.ds(c * L, L)
        g_x0[sl] = bx0
        g_y0[sl] = by0
        g_x1[sl] = bx1
        g_y1[sl] = by1
        g_ab[sl] = gw * gh
        g_w[sl] = gw
        g_hh[sl] = gh
        g_cx[sl] = bx0 + 0.5 * gw
        g_cy[sl] = by0 + 0.5 * gh

    wv = w_v[...]
    hv = h_v[...]

    def chunk_body(i, acc):
        sl = pl.ds(i * L, L)
        ax0 = a_x0[sl]
        ay0 = a_y0[sl]
        ax1 = a_x1[sl]
        ay1 = a_y1[sl]
        aw = ax1 - ax0
        ah = ay1 - ay0
        area_a = aw * ah
        inside = ((ax0 >= 0.0) & (ay0 >= 0.0) & (ax1 <= wv) & (ay1 <= hv))

        def gt_body(j, carry):
            b_iou, b_idx = carry
            jj = jnp.full((16,), j, jnp.int32)
            bx0 = plsc.load_gather(g_x0, [jj])
            by0 = plsc.load_gather(g_y0, [jj])
            bx1 = plsc.load_gather(g_x1, [jj])
            by1 = plsc.load_gather(g_y1, [jj])
            ab = plsc.load_gather(g_ab, [jj])
            wx = jnp.maximum(jnp.minimum(ax1, bx1) - jnp.maximum(ax0, bx0), 0.0)
            wy = jnp.maximum(jnp.minimum(ay1, by1) - jnp.maximum(ay0, by0), 0.0)
            inter = wx * wy
            iou = inter / ((area_a + ab) - inter)
            upd = iou > b_iou
            return (jnp.where(upd, iou, b_iou), jnp.where(upd, jj, b_idx))

        best_iou, best_idx = lax.fori_loop(
            0, G, gt_body,
            (jnp.full((16,), -1.0, jnp.float32), jnp.zeros((16,), jnp.int32)))

        neg = best_iou < NEG_IOU
        pos = best_iou >= POS_IOU
        labf = jnp.where(pos, 1.0, jnp.where(neg, 0.0, -1.0))
        lab_v[sl] = jnp.where(inside, labf, -1.0)

        bgx = plsc.load_gather(g_cx, [best_idx])
        bgy = plsc.load_gather(g_cy, [best_idx])
        bgw = plsc.load_gather(g_w, [best_idx])
        bgh = plsc.load_gather(g_hh, [best_idx])
        acx = ax0 + 0.5 * aw
        acy = ay0 + 0.5 * ah
        tx = (bgx - acx) / aw
        ty = (bgy - acy) / ah
        tw = _vlog(bgw / aw)
        th = _vlog(bgh / ah)

        ia4 = (lanes + i * L) * 4
        ones = jnp.full((16,), 1.0, jnp.float32)
        for c, tv in enumerate((tx, ty, tw, th)):
            plsc.store_scatter(tgt_v, [ia4 + c], jnp.where(inside, tv, ones))

        winv = jnp.where(inside, jnp.where(pos, 1.0, 0.0), 1.0)
        for c in range(4):
            plsc.store_scatter(win_v, [ia4 + c], winv)

        validm = inside & (neg | pos)
        code_v[sl] = jnp.where(inside, jnp.where(neg | pos, 2.0, 0.0), 1.0)
        return acc + jnp.where(validm, 1.0, 0.0)

    acc = lax.fori_loop(0, NCH, chunk_body,
                        jnp.zeros((16,), jnp.float32))
    cnt_v[...] = acc

    pltpu.sync_copy(lab_v, lab_h.at[pl.ds(base, PT)])
    pltpu.sync_copy(tgt_v, tgt_h.at[pl.ds(base * 4, PT * 4)])
    pltpu.sync_copy(win_v, win_h.at[pl.ds(base * 4, PT * 4)])
    pltpu.sync_copy(code_v, code_h.at[pl.ds(base, PT)])
    pltpu.sync_copy(cnt_v, cnt_h.at[pl.ds(wid * L, L)])


def _pass2_body(PT, NCH,
                code_h, cnt_h, wout_h,
                code_v, cnt_v, wout_v):
    wid = lax.axis_index("s") * NC + lax.axis_index("c")
    base = wid * PT

    pltpu.sync_copy(code_h.at[pl.ds(base, PT)], code_v)
    pltpu.sync_copy(cnt_h, cnt_v)

    tot = jnp.zeros((16,), jnp.float32)
    for r in range(NW):
        tot = tot + cnt_v[pl.ds(r * L, L)]
    totv = jnp.full((16,), jnp.sum(tot), jnp.float32)
    invv = jnp.full((16,), 1.0, jnp.float32) / totv

    lanes = lax.iota(jnp.int32, 16)

    def chunk_body(i, _):
        sl = pl.ds(i * L, L)
        cv = code_v[sl]
        w = jnp.where(cv == 2.0, invv,
                      jnp.where(cv == 1.0, 1.0, 0.0))
        ia4 = (lanes + i * L) * 4
        for c in range(4):
            plsc.store_scatter(wout_v, [ia4 + c], w)
        return 0

    lax.fori_loop(0, NCH, chunk_body, 0)
    pltpu.sync_copy(wout_v, wout_h.at[pl.ds(base * 4, PT * 4)])


@jax.jit
def kernel(gt_bboxes, image_shape, anchors):
    N = anchors.shape[0]
    G = gt_bboxes.shape[0]
    NP = -(-N // (NW * L)) * (NW * L)
    PT = NP // NW
    NCH = PT // L

    pad = NP - N
    cols = []
    for c in range(4):
        cols.append(jnp.pad(anchors[:, c], (0, pad), constant_values=-1.0))
    x0, y0, x1, y1 = cols
    gt_flat = gt_bboxes.reshape(-1)
    w16 = jnp.full((16,), image_shape[1], jnp.float32)
    h16 = jnp.full((16,), image_shape[0], jnp.float32)

    mesh = plsc.VectorSubcoreMesh(core_axis_name="c", subcore_axis_name="s",
                                  num_cores=NC, num_subcores=NS)
    cparams = pltpu.CompilerParams(needs_layout_passes=False)

    f32 = jnp.float32
    pass1 = pl.kernel(
        functools.partial(_pass1_body, G, PT, NCH),
        out_type=(
            jax.ShapeDtypeStruct((NP,), f32),
            jax.ShapeDtypeStruct((NP * 4,), f32),
            jax.ShapeDtypeStruct((NP * 4,), f32),
            jax.ShapeDtypeStruct((NP,), f32),
            jax.ShapeDtypeStruct((NW * L,), f32),
        ),
        mesh=mesh,
        compiler_params=cparams,
        scratch_types=(
            pltpu.VMEM((G * 4,), f32),
            pltpu.VMEM((G,), f32), pltpu.VMEM((G,), f32),
            pltpu.VMEM((G,), f32), pltpu.VMEM((G,), f32),
            pltpu.VMEM((G,), f32), pltpu.VMEM((G,), f32),
            pltpu.VMEM((G,), f32), pltpu.VMEM((G,), f32),
            pltpu.VMEM((G,), f32),
            pltpu.VMEM((PT,), f32), pltpu.VMEM((PT,), f32),
            pltpu.VMEM((PT,), f32), pltpu.VMEM((PT,), f32),
            pltpu.VMEM((PT,), f32),
            pltpu.VMEM((PT * 4,), f32),
            pltpu.VMEM((PT * 4,), f32),
            pltpu.VMEM((PT,), f32),
            pltpu.VMEM((16,), f32), pltpu.VMEM((16,), f32),
            pltpu.VMEM((16,), f32),
        ),
    )
    labels_p, targets_p, win_p, code_p, counts = pass1(
        gt_flat, w16, h16, x0, y0, x1, y1)

    pass2 = pl.kernel(
        functools.partial(_pass2_body, PT, NCH),
        out_type=jax.ShapeDtypeStruct((NP * 4,), f32),
        mesh=mesh,
        compiler_params=cparams,
        scratch_types=(
            pltpu.VMEM((PT,), f32),
            pltpu.VMEM((NW * L,), f32),
            pltpu.VMEM((PT * 4,), f32),
        ),
    )
    wout_p = pass2(code_p, counts)

    return (labels_p[:N],
            targets_p.reshape(NP, 4)[:N],
            win_p.reshape(NP, 4)[:N],
            wout_p.reshape(NP, 4)[:N])
